# baseline (speedup 1.0000x reference)
import functools

import jax
import jax.numpy as jnp
from jax import lax
from jax.experimental import pallas as pl
from jax.experimental.pallas import tpu as pltpu
from jax.experimental.pallas import tpu_sc as plsc

NC = 2
NS = 16
NW = NC * NS
L = 16

G = 125


def _worker_id():
  return lax.axis_index("s") * NC + lax.axis_index("c")


def _make_deg_kernel(n_nodes, edges_per_worker):
  T = edges_per_worker
  mesh = plsc.VectorSubcoreMesh(core_axis_name="c", subcore_axis_name="s")

  @functools.partial(
      pl.kernel,
      out_type=(
          jax.ShapeDtypeStruct((NW, n_nodes), jnp.float32),
          jax.ShapeDtypeStruct((NW, n_nodes), jnp.float32),
      ),
      mesh=mesh,
      scratch_types=[
          pltpu.VMEM((T,), jnp.int32),
          pltpu.VMEM((n_nodes,), jnp.float32),
      ],
      compiler_params=pltpu.CompilerParams(needs_layout_passes=False),
  )
  def k(src_hbm, dst_hbm, outs_hbm, outd_hbm, idx_v, hist_v):
    wid = _worker_id()
    zeros = jnp.zeros((L,), jnp.float32)
    ones = jnp.ones((L,), jnp.float32)

    def run(idx_hbm, out_hbm):
      def zstep(i, _):
        hist_v[pl.ds(i * L, L)] = zeros
        return 0

      lax.fori_loop(0, n_nodes // L, zstep, 0)
      pltpu.sync_copy(idx_hbm.at[wid], idx_v)

      def astep(i, _):
        iv = idx_v[pl.ds(i * L, L)]
        plsc.addupdate_scatter(hist_v, [iv], ones)
        return 0

      lax.fori_loop(0, T // L, astep, 0)
      pltpu.sync_copy(hist_v, out_hbm.at[wid])

    run(src_hbm, outs_hbm)
    run(dst_hbm, outd_hbm)

  return k


def _make_gs_kernel(n_acc, n_drain, n_chunks, d):
  rpt = (n_drain // NS) // 8 * 8
  rem_start = rpt * NS
  rem = n_drain - rem_start
  assert rem % 8 == 0
  nb = 7
  nbuf = 8
  mesh = plsc.VectorSubcoreMesh(core_axis_name="c", subcore_axis_name="s")

  @functools.partial(
      pl.kernel,
      out_type=jax.ShapeDtypeStruct((NC, n_drain, d), jnp.float32),
      mesh=mesh,
      scratch_types=[
          pltpu.VMEM((n_chunks, G), jnp.int32),
          pltpu.VMEM((n_chunks, G), jnp.int32),
          pltpu.VMEM((nbuf, G, d), jnp.float32),
          pltpu.VMEM_SHARED((n_acc, d), jnp.float32),
          [pltpu.SemaphoreType.DMA] * nbuf,
          [pltpu.SemaphoreType.DMA] * nbuf,
      ],
      compiler_params=pltpu.CompilerParams(use_tc_tiling_on_sc=False),
  )
  def k(h_hbm, sidx_hbm, didx_hbm, zero_hbm, out_hbm, sidx_v, didx_v, buf_v,
        acc_sh, gsems, ssems):
    cid = lax.axis_index("c")
    sid = lax.axis_index("s")
    wid = _worker_id()

    pltpu.sync_copy(sidx_hbm.at[wid], sidx_v)
    pltpu.sync_copy(didx_hbm.at[wid], didx_v)

    for b in range(nb):
      pltpu.async_copy(h_hbm.at[sidx_v.at[b]], buf_v.at[b], gsems[b])

    base = pl.multiple_of(sid * rpt, 8)
    pltpu.sync_copy(zero_hbm.at[pl.ds(base, rpt)], acc_sh.at[pl.ds(base, rpt)])
    if rem:
      @pl.when(sid == NS - 1)
      def _():
        pltpu.sync_copy(zero_hbm.at[pl.ds(rem_start, rem)],
                        acc_sh.at[pl.ds(rem_start, rem)])
    plsc.subcore_barrier()

    assert n_chunks % nbuf == 0

    def body(g, _):
      for b in range(nbuf):
        j = g * nbuf + b
        bp = (b + nbuf - 1) % nbuf
        bn = (b + nb) % nbuf
        pltpu.make_async_copy(h_hbm.at[sidx_v.at[j]], buf_v.at[b],
                              gsems[b]).wait()

        @pl.when(j >= 1)
        def _():
          pltpu.make_async_copy(buf_v.at[bp], acc_sh.at[didx_v.at[j - 1]],
                                ssems[bp]).wait()

        @pl.when(j + nb < n_chunks)
        def _():
          pltpu.async_copy(h_hbm.at[sidx_v.at[j + nb]], buf_v.at[bn],
                           gsems[bn])

        pltpu.async_copy(buf_v.at[b], acc_sh.at[didx_v.at[j]], ssems[b],
                         add=True)
      return 0

    lax.fori_loop(0, n_chunks // nbuf, body, 0)
    j = n_chunks - 1
    pltpu.make_async_copy(buf_v.at[j % nbuf], acc_sh.at[didx_v.at[j]],
                          ssems[j % nbuf]).wait()
    plsc.subcore_barrier()

    pltpu.sync_copy(acc_sh.at[pl.ds(base, rpt)],
                    out_hbm.at[cid, pl.ds(base, rpt)])
    if rem:
      @pl.when(sid == NS - 1)
      def _():
        pltpu.sync_copy(acc_sh.at[pl.ds(rem_start, rem)],
                        out_hbm.at[cid, pl.ds(rem_start, rem)])

  return k


def _mm1_body(x_ref, w_ref, ds_ref, dd_ref, ha_ref, hb_ref, ns_ref, nd_ref):
  h = w_ref.shape[1]
  ns = lax.rsqrt(jnp.maximum(jnp.sum(ds_ref[...], axis=0), 1.0))[:, None]
  nd = lax.rsqrt(jnp.maximum(jnp.sum(dd_ref[...], axis=0), 1.0))[:, None]
  ns_ref[...] = ns
  nd_ref[...] = nd
  full = jnp.dot(x_ref[...] * ns, w_ref[...],
                 preferred_element_type=jnp.float32)
  ha_ref[...] = full[:, : h // 2]
  hb_ref[...] = full[:, h // 2 :]


def _mm1(x, w1, degs, degd):
  n, _ = x.shape
  h = w1.shape[1]
  return pl.pallas_call(
      _mm1_body,
      out_shape=[
          jax.ShapeDtypeStruct((n, h // 2), jnp.float32),
          jax.ShapeDtypeStruct((n, h // 2), jnp.float32),
          jax.ShapeDtypeStruct((n, 1), jnp.float32),
          jax.ShapeDtypeStruct((n, 1), jnp.float32),
      ],
  )(x, w1, degs, degd)


def _mm2_body(pa_ref, pb_ref, ns_ref, nd_ref, w_ref, o_ref):
  h = w_ref.shape[0]
  nd = nd_ref[...]
  ns = ns_ref[...]
  h1a = jnp.maximum((pa_ref[0] + pa_ref[1]) * nd, 0.0) * ns
  h1b = jnp.maximum((pb_ref[0] + pb_ref[1]) * nd, 0.0) * ns
  o_ref[...] = (
      jnp.dot(h1a, w_ref[: h // 2], preferred_element_type=jnp.float32)
      + jnp.dot(h1b, w_ref[h // 2 :], preferred_element_type=jnp.float32))


def _mm2(pa, pb, ns, nd, w2, blk):
  n = pa.shape[1]
  hh = pa.shape[2]
  o = w2.shape[1]
  return pl.pallas_call(
      _mm2_body,
      grid=(n // blk,),
      in_specs=[
          pl.BlockSpec((NC, blk, hh), lambda i: (0, i, 0)),
          pl.BlockSpec((NC, blk, hh), lambda i: (0, i, 0)),
          pl.BlockSpec((blk, 1), lambda i: (i, 0)),
          pl.BlockSpec((blk, 1), lambda i: (i, 0)),
          pl.BlockSpec((2 * hh, o), lambda i: (0, 0)),
      ],
      out_specs=pl.BlockSpec((blk, o), lambda i: (i, 0)),
      out_shape=jax.ShapeDtypeStruct((n, o), jnp.float32),
  )(pa, pb, ns, nd, w2)


def _fin_body(p_ref, nd_ref, o_ref):
  o_ref[...] = (p_ref[0] + p_ref[1]) * nd_ref[...]


def _fin(p, nd, blk):
  n = p.shape[1]
  o = p.shape[2]
  return pl.pallas_call(
      _fin_body,
      grid=(n // blk,),
      in_specs=[
          pl.BlockSpec((NC, blk, o), lambda i: (0, i, 0)),
          pl.BlockSpec((blk, 1), lambda i: (i, 0)),
      ],
      out_specs=pl.BlockSpec((blk, o), lambda i: (i, 0)),
      out_shape=jax.ShapeDtypeStruct((n, o), jnp.float32),
  )(p, nd)


@jax.jit
def kernel(x, edge_index, W1, W2):
  n, f = x.shape
  h = W1.shape[1]
  o = W2.shape[1]
  e = edge_index.shape[1]

  t = e // NW
  ch = t // G
  n_acc = n
  assert e % (NW * G) == 0 and ch % 4 == 0 and t % L == 0 and n % L == 0

  ei = edge_index.astype(jnp.int32)
  src_p = ei[0].reshape(NW, ch, G)
  dst_p = ei[1].reshape(NW, ch, G)
  src_t = ei[0].reshape(NW, t)
  dst_t = ei[1].reshape(NW, t)

  degs, degd = _make_deg_kernel(n, t)(src_t, dst_t)
  ha, hb, ns, nd = _mm1(x, W1, degs, degd)

  zero_h = jnp.zeros((n, h // 2), jnp.float32)
  gs = _make_gs_kernel(n_acc, n, ch, h // 2)
  p1a = gs(ha, src_p, dst_p, zero_h)
  p1b = gs(hb, src_p, dst_p, zero_h)
  h2 = _mm2(p1a, p1b, ns, nd, W2, 2000)

  zero_o = jnp.zeros((n, o), jnp.float32)
  p2 = _make_gs_kernel(n_acc, n, ch, o)(h2, src_p, dst_p, zero_o)
  return _fin(p2, nd, 2000)


# speedup vs baseline: 1.1249x; 1.1249x over previous
import functools

import jax
import jax.numpy as jnp
from jax import lax
from jax.experimental import pallas as pl
from jax.experimental.pallas import tpu as pltpu
from jax.experimental.pallas import tpu_sc as plsc

NC = 2
NS = 16
NW = NC * NS
L = 16

G = 125


def _worker_id():
  return lax.axis_index("s") * NC + lax.axis_index("c")


def _make_deg_kernel(n_nodes, edges_per_worker):
  T = edges_per_worker
  mesh = plsc.VectorSubcoreMesh(core_axis_name="c", subcore_axis_name="s")

  @functools.partial(
      pl.kernel,
      out_type=(
          jax.ShapeDtypeStruct((NW, n_nodes), jnp.float32),
          jax.ShapeDtypeStruct((NW, n_nodes), jnp.float32),
      ),
      mesh=mesh,
      scratch_types=[
          pltpu.VMEM((T,), jnp.int32),
          pltpu.VMEM((n_nodes,), jnp.float32),
      ],
      compiler_params=pltpu.CompilerParams(needs_layout_passes=False),
  )
  def k(src_hbm, dst_hbm, outs_hbm, outd_hbm, idx_v, hist_v):
    wid = _worker_id()
    zeros = jnp.zeros((L,), jnp.float32)
    ones = jnp.ones((L,), jnp.float32)

    def run(idx_hbm, out_hbm):
      def zstep(i, _):
        hist_v[pl.ds(i * L, L)] = zeros
        return 0

      lax.fori_loop(0, n_nodes // L, zstep, 0)
      pltpu.sync_copy(idx_hbm.at[wid], idx_v)

      def astep(i, _):
        iv = idx_v[pl.ds(i * L, L)]
        plsc.addupdate_scatter(hist_v, [iv], ones)
        return 0

      lax.fori_loop(0, T // L, astep, 0)
      pltpu.sync_copy(hist_v, out_hbm.at[wid])

    run(src_hbm, outs_hbm)
    run(dst_hbm, outd_hbm)

  return k


def _make_gs_kernel(n_acc, n_drain, n_chunks, d):
  rpt = (n_drain // NS) // 8 * 8
  rem_start = rpt * NS
  rem = n_drain - rem_start
  assert rem % 8 == 0
  nb = 3
  nbuf = 4
  mesh = plsc.VectorSubcoreMesh(core_axis_name="c", subcore_axis_name="s")

  @functools.partial(
      pl.kernel,
      out_type=jax.ShapeDtypeStruct((NC, n_drain, d), jnp.float32),
      mesh=mesh,
      scratch_types=[
          pltpu.VMEM((n_chunks, G), jnp.int32),
          pltpu.VMEM((n_chunks, G), jnp.int32),
          pltpu.VMEM((nbuf, G, d), jnp.float32),
          pltpu.VMEM_SHARED((n_acc, d), jnp.float32),
          [pltpu.SemaphoreType.DMA] * nbuf,
          [pltpu.SemaphoreType.DMA] * nbuf,
      ],
      compiler_params=pltpu.CompilerParams(use_tc_tiling_on_sc=False),
  )
  def k(h_hbm, sidx_hbm, didx_hbm, zero_hbm, out_hbm, sidx_v, didx_v, buf_v,
        acc_sh, gsems, ssems):
    cid = lax.axis_index("c")
    sid = lax.axis_index("s")
    wid = _worker_id()

    pltpu.sync_copy(sidx_hbm.at[wid], sidx_v)
    pltpu.sync_copy(didx_hbm.at[wid], didx_v)

    for b in range(nb):
      pltpu.async_copy(h_hbm.at[sidx_v.at[b]], buf_v.at[b], gsems[b])

    base = pl.multiple_of(sid * rpt, 8)
    pltpu.sync_copy(zero_hbm.at[pl.ds(base, rpt)], acc_sh.at[pl.ds(base, rpt)])
    if rem:
      @pl.when(sid == NS - 1)
      def _():
        pltpu.sync_copy(zero_hbm.at[pl.ds(rem_start, rem)],
                        acc_sh.at[pl.ds(rem_start, rem)])
    plsc.subcore_barrier()

    assert n_chunks % nbuf == 0

    def body(g, _):
      for b in range(nbuf):
        j = g * nbuf + b
        bp = (b + nbuf - 1) % nbuf
        bn = (b + nb) % nbuf
        pltpu.make_async_copy(h_hbm.at[sidx_v.at[j]], buf_v.at[b],
                              gsems[b]).wait()

        @pl.when(j >= 1)
        def _():
          pltpu.make_async_copy(buf_v.at[bp], acc_sh.at[didx_v.at[j - 1]],
                                ssems[bp]).wait()

        @pl.when(j + nb < n_chunks)
        def _():
          pltpu.async_copy(h_hbm.at[sidx_v.at[j + nb]], buf_v.at[bn],
                           gsems[bn])

        pltpu.async_copy(buf_v.at[b], acc_sh.at[didx_v.at[j]], ssems[b],
                         add=True)
      return 0

    lax.fori_loop(0, n_chunks // nbuf, body, 0)
    j = n_chunks - 1
    pltpu.make_async_copy(buf_v.at[j % nbuf], acc_sh.at[didx_v.at[j]],
                          ssems[j % nbuf]).wait()
    plsc.subcore_barrier()

    pltpu.sync_copy(acc_sh.at[pl.ds(base, rpt)],
                    out_hbm.at[cid, pl.ds(base, rpt)])
    if rem:
      @pl.when(sid == NS - 1)
      def _():
        pltpu.sync_copy(acc_sh.at[pl.ds(rem_start, rem)],
                        out_hbm.at[cid, pl.ds(rem_start, rem)])

  return k


def _mm1_body(x_ref, w_ref, ds_ref, dd_ref, ha_ref, hb_ref, ns_ref, nd_ref):
  h = w_ref.shape[1]
  ns = lax.rsqrt(jnp.maximum(jnp.sum(ds_ref[...], axis=0), 1.0))[:, None]
  nd = lax.rsqrt(jnp.maximum(jnp.sum(dd_ref[...], axis=0), 1.0))[:, None]
  ns_ref[...] = ns
  nd_ref[...] = nd
  full = jnp.dot(x_ref[...] * ns, w_ref[...],
                 preferred_element_type=jnp.float32)
  ha_ref[...] = full[:, : h // 2]
  hb_ref[...] = full[:, h // 2 :]


def _mm1(x, w1, degs, degd):
  n, _ = x.shape
  h = w1.shape[1]
  return pl.pallas_call(
      _mm1_body,
      out_shape=[
          jax.ShapeDtypeStruct((n, h // 2), jnp.float32),
          jax.ShapeDtypeStruct((n, h // 2), jnp.float32),
          jax.ShapeDtypeStruct((n, 1), jnp.float32),
          jax.ShapeDtypeStruct((n, 1), jnp.float32),
      ],
  )(x, w1, degs, degd)


def _mm2_body(pa_ref, pb_ref, ns_ref, nd_ref, w_ref, o_ref):
  hh = w_ref.shape[0] // 2
  nd = nd_ref[...]
  ns = ns_ref[...]
  h1a = jnp.maximum((pa_ref[0] + pa_ref[1]) * nd, 0.0) * ns
  h1b = jnp.maximum((pb_ref[0] + pb_ref[1]) * nd, 0.0) * ns
  w = w_ref[...]
  z = jnp.zeros((hh, hh), jnp.float32)
  bda = jnp.concatenate(
      [jnp.concatenate([w[:hh], z], 1), jnp.concatenate([z, w[:hh]], 1)], 0)
  bdb = jnp.concatenate(
      [jnp.concatenate([w[hh:], z], 1), jnp.concatenate([z, w[hh:]], 1)], 0)
  o_ref[...] = (jnp.dot(h1a, bda, preferred_element_type=jnp.float32)
                + jnp.dot(h1b, bdb, preferred_element_type=jnp.float32))


def _mm2(pa, pb, ns, nd, w2, blk):
  np_ = pa.shape[1]
  w = pa.shape[2]
  o = w2.shape[1]
  return pl.pallas_call(
      _mm2_body,
      grid=(np_ // blk,),
      in_specs=[
          pl.BlockSpec((NC, blk, w), lambda i: (0, i, 0)),
          pl.BlockSpec((NC, blk, w), lambda i: (0, i, 0)),
          pl.BlockSpec((blk, w), lambda i: (i, 0)),
          pl.BlockSpec((blk, w), lambda i: (i, 0)),
          pl.BlockSpec((2 * o, o), lambda i: (0, 0)),
      ],
      out_specs=pl.BlockSpec((blk, w), lambda i: (i, 0)),
      out_shape=jax.ShapeDtypeStruct((np_, w), jnp.float32),
  )(pa, pb, ns, nd, w2)


def _fin_body(p_ref, nd_ref, o_ref):
  o_ref[...] = (p_ref[0] + p_ref[1]) * nd_ref[...]


def _fin(p, nd, blk):
  np_ = p.shape[1]
  w = p.shape[2]
  return pl.pallas_call(
      _fin_body,
      grid=(np_ // blk,),
      in_specs=[
          pl.BlockSpec((NC, blk, w), lambda i: (0, i, 0)),
          pl.BlockSpec((blk, w), lambda i: (i, 0)),
      ],
      out_specs=pl.BlockSpec((blk, w), lambda i: (i, 0)),
      out_shape=jax.ShapeDtypeStruct((np_, w), jnp.float32),
  )(p, nd)


@jax.jit
def kernel(x, edge_index, W1, W2):
  n, f = x.shape
  h = W1.shape[1]
  o = W2.shape[1]
  e = edge_index.shape[1]

  t = e // NW
  ch = t // G
  n_acc = n
  assert e % (NW * G) == 0 and ch % 4 == 0 and t % L == 0 and n % L == 0

  ei = edge_index.astype(jnp.int32)
  src_p = ei[0].reshape(NW, ch, G)
  dst_p = ei[1].reshape(NW, ch, G)
  src_t = ei[0].reshape(NW, t)
  dst_t = ei[1].reshape(NW, t)

  degs, degd = _make_deg_kernel(n, t)(src_t, dst_t)
  ha, hb, ns, nd = _mm1(x, W1, degs, degd)
  ns_p = jnp.broadcast_to(ns, (n, h // 2)).reshape(n // 2, h)
  nd_p = jnp.broadcast_to(nd, (n, h // 2)).reshape(n // 2, h)

  zero_h = jnp.zeros((n, h // 2), jnp.float32)
  gs = _make_gs_kernel(n_acc, n, ch, h // 2)
  p1a = gs(ha, src_p, dst_p, zero_h)
  p1b = gs(hb, src_p, dst_p, zero_h)
  h2_p = _mm2(p1a.reshape(NC, n // 2, h), p1b.reshape(NC, n // 2, h),
              ns_p, nd_p, W2, 1000)

  zero_o = jnp.zeros((n, o), jnp.float32)
  p2 = _make_gs_kernel(n_acc, n, ch, o)(h2_p.reshape(n, o), src_p, dst_p,
                                        zero_o)
  out_p = _fin(p2.reshape(NC, n // 2, 2 * o), nd_p, 1000)
  return out_p.reshape(n, o)


# speedup vs baseline: 1.1381x; 1.0118x over previous
import functools

import jax
import jax.numpy as jnp
from jax import lax
from jax.experimental import pallas as pl
from jax.experimental.pallas import tpu as pltpu
from jax.experimental.pallas import tpu_sc as plsc

NC = 2
NS = 16
NW = NC * NS
L = 16

G = 125


def _worker_id():
  return lax.axis_index("s") * NC + lax.axis_index("c")


def _make_deg_kernel(n_nodes, edges_per_worker):
  T = edges_per_worker
  mesh = plsc.VectorSubcoreMesh(core_axis_name="c", subcore_axis_name="s")

  @functools.partial(
      pl.kernel,
      out_type=(
          jax.ShapeDtypeStruct((NW, n_nodes), jnp.float32),
          jax.ShapeDtypeStruct((NW, n_nodes), jnp.float32),
      ),
      mesh=mesh,
      scratch_types=[
          pltpu.VMEM((T,), jnp.int32),
          pltpu.VMEM((n_nodes,), jnp.float32),
      ],
      compiler_params=pltpu.CompilerParams(
          needs_layout_passes=False, use_tc_tiling_on_sc=False),
  )
  def k(ei_hbm, outs_hbm, outd_hbm, idx_v, hist_v):
    wid = _worker_id()
    zeros = jnp.zeros((L,), jnp.float32)
    ones = jnp.ones((L,), jnp.float32)

    def run(row, out_hbm):
      def zstep(i, _):
        hist_v[pl.ds(i * L, L)] = zeros
        return 0

      lax.fori_loop(0, n_nodes // L, zstep, 0)
      pltpu.sync_copy(ei_hbm.at[row, pl.ds(wid * T, T)], idx_v)

      def astep(i, _):
        iv = idx_v[pl.ds(i * L, L)]
        plsc.addupdate_scatter(hist_v, [iv], ones)
        return 0

      lax.fori_loop(0, T // L, astep, 0)
      pltpu.sync_copy(hist_v, out_hbm.at[wid])

    run(0, outs_hbm)
    run(1, outd_hbm)

  return k


def _make_gs_kernel(n_acc, n_drain, n_chunks, d):
  rpt = (n_drain // NS) // 8 * 8
  rem_start = rpt * NS
  rem = n_drain - rem_start
  assert rem % 8 == 0
  nb = 3
  nbuf = 4
  mesh = plsc.VectorSubcoreMesh(core_axis_name="c", subcore_axis_name="s")

  @functools.partial(
      pl.kernel,
      out_type=jax.ShapeDtypeStruct((NC, n_drain, d), jnp.float32),
      mesh=mesh,
      scratch_types=[
          pltpu.VMEM((n_chunks, G), jnp.int32),
          pltpu.VMEM((n_chunks, G), jnp.int32),
          pltpu.VMEM((nbuf, G, d), jnp.float32),
          pltpu.VMEM_SHARED((n_acc, d), jnp.float32),
          [pltpu.SemaphoreType.DMA] * nbuf,
          [pltpu.SemaphoreType.DMA] * nbuf,
      ],
      compiler_params=pltpu.CompilerParams(use_tc_tiling_on_sc=False),
  )
  def k(h_hbm, sidx_hbm, didx_hbm, zero_hbm, out_hbm, sidx_v, didx_v, buf_v,
        acc_sh, gsems, ssems):
    cid = lax.axis_index("c")
    sid = lax.axis_index("s")
    wid = _worker_id()

    pltpu.sync_copy(sidx_hbm.at[wid], sidx_v)
    pltpu.sync_copy(didx_hbm.at[wid], didx_v)

    for b in range(nb):
      pltpu.async_copy(h_hbm.at[sidx_v.at[b]], buf_v.at[b], gsems[b])

    base = pl.multiple_of(sid * rpt, 8)
    pltpu.sync_copy(zero_hbm.at[pl.ds(base, rpt)], acc_sh.at[pl.ds(base, rpt)])
    if rem:
      @pl.when(sid == NS - 1)
      def _():
        pltpu.sync_copy(zero_hbm.at[pl.ds(rem_start, rem)],
                        acc_sh.at[pl.ds(rem_start, rem)])
    plsc.subcore_barrier()

    assert n_chunks % nbuf == 0

    def body(g, _):
      for b in range(nbuf):
        j = g * nbuf + b
        bp = (b + nbuf - 1) % nbuf
        bn = (b + nb) % nbuf
        pltpu.make_async_copy(h_hbm.at[sidx_v.at[j]], buf_v.at[b],
                              gsems[b]).wait()

        @pl.when(j >= 1)
        def _():
          pltpu.make_async_copy(buf_v.at[bp], acc_sh.at[didx_v.at[j - 1]],
                                ssems[bp]).wait()

        @pl.when(j + nb < n_chunks)
        def _():
          pltpu.async_copy(h_hbm.at[sidx_v.at[j + nb]], buf_v.at[bn],
                           gsems[bn])

        pltpu.async_copy(buf_v.at[b], acc_sh.at[didx_v.at[j]], ssems[b],
                         add=True)
      return 0

    lax.fori_loop(0, n_chunks // nbuf, body, 0)
    j = n_chunks - 1
    pltpu.make_async_copy(buf_v.at[j % nbuf], acc_sh.at[didx_v.at[j]],
                          ssems[j % nbuf]).wait()
    plsc.subcore_barrier()

    pltpu.sync_copy(acc_sh.at[pl.ds(base, rpt)],
                    out_hbm.at[cid, pl.ds(base, rpt)])
    if rem:
      @pl.when(sid == NS - 1)
      def _():
        pltpu.sync_copy(acc_sh.at[pl.ds(rem_start, rem)],
                        out_hbm.at[cid, pl.ds(rem_start, rem)])

  return k


def _mm1_body(x_ref, w_ref, ds_ref, dd_ref, ha_ref, hb_ref, ns_ref, nd_ref):
  h = w_ref.shape[1]
  ns = lax.rsqrt(jnp.maximum(jnp.sum(ds_ref[...], axis=0), 1.0))[:, None]
  nd = lax.rsqrt(jnp.maximum(jnp.sum(dd_ref[...], axis=0), 1.0))[:, None]
  ns_ref[...] = ns
  nd_ref[...] = nd
  full = jnp.dot(x_ref[...] * ns, w_ref[...],
                 preferred_element_type=jnp.float32)
  ha_ref[...] = full[:, : h // 2]
  hb_ref[...] = full[:, h // 2 :]


def _mm1(x, w1, degs, degd):
  n, _ = x.shape
  h = w1.shape[1]
  return pl.pallas_call(
      _mm1_body,
      out_shape=[
          jax.ShapeDtypeStruct((n, h // 2), jnp.float32),
          jax.ShapeDtypeStruct((n, h // 2), jnp.float32),
          jax.ShapeDtypeStruct((n, 1), jnp.float32),
          jax.ShapeDtypeStruct((n, 1), jnp.float32),
      ],
  )(x, w1, degs, degd)


def _mm2_body(pa_ref, pb_ref, ns_ref, nd_ref, w_ref, o_ref):
  hh = w_ref.shape[0] // 2
  nd = nd_ref[...]
  ns = ns_ref[...]
  h1a = jnp.maximum((pa_ref[0] + pa_ref[1]) * nd, 0.0) * ns
  h1b = jnp.maximum((pb_ref[0] + pb_ref[1]) * nd, 0.0) * ns
  w = w_ref[...]
  z = jnp.zeros((hh, hh), jnp.float32)
  bda = jnp.concatenate(
      [jnp.concatenate([w[:hh], z], 1), jnp.concatenate([z, w[:hh]], 1)], 0)
  bdb = jnp.concatenate(
      [jnp.concatenate([w[hh:], z], 1), jnp.concatenate([z, w[hh:]], 1)], 0)
  o_ref[...] = (jnp.dot(h1a, bda, preferred_element_type=jnp.float32)
                + jnp.dot(h1b, bdb, preferred_element_type=jnp.float32))


def _mm2(pa, pb, ns, nd, w2, blk):
  np_ = pa.shape[1]
  w = pa.shape[2]
  o = w2.shape[1]
  return pl.pallas_call(
      _mm2_body,
      grid=(np_ // blk,),
      in_specs=[
          pl.BlockSpec((NC, blk, w), lambda i: (0, i, 0)),
          pl.BlockSpec((NC, blk, w), lambda i: (0, i, 0)),
          pl.BlockSpec((blk, w), lambda i: (i, 0)),
          pl.BlockSpec((blk, w), lambda i: (i, 0)),
          pl.BlockSpec((2 * o, o), lambda i: (0, 0)),
      ],
      out_specs=pl.BlockSpec((blk, w), lambda i: (i, 0)),
      out_shape=jax.ShapeDtypeStruct((np_, w), jnp.float32),
  )(pa, pb, ns, nd, w2)


def _fin_body(p_ref, nd_ref, o_ref):
  o_ref[...] = (p_ref[0] + p_ref[1]) * nd_ref[...]


def _fin(p, nd, blk):
  np_ = p.shape[1]
  w = p.shape[2]
  return pl.pallas_call(
      _fin_body,
      grid=(np_ // blk,),
      in_specs=[
          pl.BlockSpec((NC, blk, w), lambda i: (0, i, 0)),
          pl.BlockSpec((blk, w), lambda i: (i, 0)),
      ],
      out_specs=pl.BlockSpec((blk, w), lambda i: (i, 0)),
      out_shape=jax.ShapeDtypeStruct((np_, w), jnp.float32),
  )(p, nd)


@jax.jit
def kernel(x, edge_index, W1, W2):
  n, f = x.shape
  h = W1.shape[1]
  o = W2.shape[1]
  e = edge_index.shape[1]

  t = e // NW
  ch = t // G
  n_acc = n
  assert e % (NW * G) == 0 and ch % 4 == 0 and t % L == 0 and n % L == 0

  ei = edge_index.astype(jnp.int32)
  src_p = ei[0].reshape(NW, ch, G)
  dst_p = ei[1].reshape(NW, ch, G)

  degs, degd = _make_deg_kernel(n, t)(ei)
  ha, hb, ns, nd = _mm1(x, W1, degs, degd)
  ns_p = jnp.broadcast_to(ns, (n, h // 2)).reshape(n // 2, h)
  nd_p = jnp.broadcast_to(nd, (n, h // 2)).reshape(n // 2, h)

  zero_h = jnp.zeros((n, h // 2), jnp.float32)
  gs = _make_gs_kernel(n_acc, n, ch, h // 2)
  p1a = gs(ha, src_p, dst_p, zero_h)
  p1b = gs(hb, src_p, dst_p, zero_h)
  h2_p = _mm2(p1a.reshape(NC, n // 2, h), p1b.reshape(NC, n // 2, h),
              ns_p, nd_p, W2, 1000)

  zero_o = jnp.zeros((n, o), jnp.float32)
  p2 = _make_gs_kernel(n_acc, n, ch, o)(h2_p.reshape(n, o), src_p, dst_p,
                                        zero_o)
  out_p = _fin(p2.reshape(NC, n // 2, 2 * o), nd_p, 1000)
  return out_p.reshape(n, o)
